# baseline (speedup 1.0000x reference)
import jax
import jax.numpy as jnp
from jax import lax
from jax.experimental import pallas as pl
from jax.experimental.pallas import tpu as pltpu
from jax.experimental.pallas import tpu_sc as plsc

N = 10000
E = 320000
D = 128
HALF = 64
T = 4
K = 10
ALPHA = 0.1

NC = 2
NS = 16
LANES = 16
CHUNK = 128
NP = 10240
NCH = 160
GRP = 8
NSG = NCH // GRP
HG = 2
EPT = NCH * CHUNK
E_PAD = NS * EPT
ROWS_PT = NP // NS
RCH = 128
CCH = 64


def _tc_pre(x_ref, w_ref, b_ref, m_ref, ts_ref, t01_ref, mean_ref, std_ref):
    x = x_ref[...]
    w = w_ref[...]
    b = b_ref[...]
    mask = m_ref[...]
    h = jnp.dot(x, w, preferred_element_type=jnp.float32,
                precision=lax.Precision.HIGHEST) + b[None, :]
    nrm = jnp.sqrt(jnp.sum(h * h, axis=1, keepdims=True))
    h = h / jnp.maximum(nrm, 1e-12)
    counts = jnp.sum(mask, axis=1)
    inv_c = 1.0 / counts
    means = jnp.dot(mask, h, preferred_element_type=jnp.float32,
                    precision=lax.Precision.HIGHEST) * inv_c[:, None]
    m2 = jnp.dot(mask, h * h, preferred_element_type=jnp.float32,
                 precision=lax.Precision.HIGHEST) * inv_c[:, None]
    var = m2 - means * means
    std = jnp.sqrt(jnp.maximum(var, 0.0))
    std = std * jnp.sqrt(counts)[:, None] + 1e-9
    mu = jnp.zeros((N, D), jnp.float32)
    sg = jnp.zeros((N, D), jnp.float32)
    for t in range(T):
        mt = mask[t][:, None]
        mu = mu + mt * means[t][None, :]
        sg = sg + mt * std[t][None, :]
    tilde = (h - mu) / sg
    pad = jnp.zeros((NP - N, HALF), jnp.float32)
    for half in range(2):
        th = tilde[:, half * HALF:(half + 1) * HALF]
        ts_ref[half, pl.ds(0, N)] = th
        ts_ref[half, pl.ds(N, NP - N)] = pad
        t01_ref[half, pl.ds(0, N)] = ALPHA * th
        t01_ref[half, pl.ds(N, NP - N)] = pad
    mean_ref[...] = means
    std_ref[...] = std


def _tc_post(z_ref, m_ref, mean_ref, std_ref, w_ref, b_ref, o_ref):
    mask = m_ref[...]
    means = mean_ref[...]
    std = std_ref[...]
    mu = jnp.zeros((N, D), jnp.float32)
    sg = jnp.zeros((N, D), jnp.float32)
    for t in range(T):
        mt = mask[t][:, None]
        mu = mu + mt * means[t][None, :]
        sg = sg + mt * std[t][None, :]
    z = jnp.concatenate([z_ref[0, pl.ds(0, N)], z_ref[1, pl.ds(0, N)]], axis=1)
    z = z * sg + mu
    o = jnp.dot(z, w_ref[...], preferred_element_type=jnp.float32,
                precision=lax.Precision.HIGHEST) + b_ref[...][None, :]
    nrm = jnp.sqrt(jnp.sum(o * o, axis=1, keepdims=True))
    o_ref[...] = o / jnp.maximum(nrm, 1e-12)


def _fill_rows(ref, nrows, ncols16, value):
    v = jnp.full((LANES,), value, jnp.float32)

    @pl.loop(0, nrows)
    def _(i):
        for m in range(ncols16):
            ref[i, pl.ds(m * LANES, LANES)] = v


def _sc_diffuse(ts_hbm, t01_hbm, srcb_hbm, dstb_hbm,
                zout_hbm, t01d_hbm,
                sidx0, sidx1, didx0, didx1, bk0, bk1, bk2, bk3, abuf,
                ztab, acc, s9smem, sinvsmem,
                gsemA, gsemB, gsemC, gsemD, ssemA, ssemB, ssemC, ssemD, isem):
    c = lax.axis_index("c")
    s = lax.axis_index("s")
    cN = c * NP
    r0 = s * ROWS_PT
    g0 = s * NSG

    def zero_acc():
        _fill_rows(bk0, RCH, HALF // LANES, 0.0)
        for q in range(ROWS_PT // RCH):
            pltpu.sync_copy(bk0, acc.at[pl.ds(r0 + q * RCH, RCH), :])
        plsc.subcore_barrier()

    for q in range(ROWS_PT // RCH):
        rq = r0 + q * RCH
        pltpu.sync_copy(ts_hbm.at[pl.ds(cN + rq, RCH), :], abuf)
        pltpu.sync_copy(abuf, ztab.at[pl.ds(rq, RCH), :])
        pltpu.sync_copy(abuf, zout_hbm.at[pl.ds(cN + rq, RCH), :])

    zero_acc()
    _fill_rows(bk0, RCH, HALF // LANES, 1.0)

    @pl.loop(0, NSG)
    def _(t):
        pltpu.sync_copy(dstb_hbm.at[g0 + t], didx0)
        for b in range(GRP):
            pltpu.sync_copy(bk0, acc.at[didx0.at[b]], add=True)

    plsc.subcore_barrier()
    for q in range(ROWS_PT // RCH):
        pltpu.sync_copy(acc.at[pl.ds(r0 + q * RCH, RCH), :], abuf)

        @pl.loop(0, RCH)
        def _(i):
            v = jnp.maximum(abuf[i, pl.ds(0, LANES)], 1.0)
            s9smem[q * RCH + i] = jnp.max((1.0 - ALPHA) / v)
            sinvsmem[q * RCH + i] = jnp.max(v * (1.0 / (1.0 - ALPHA)))

    for q in range(ROWS_PT // RCH):
        rq = r0 + q * RCH
        pltpu.sync_copy(t01_hbm.at[pl.ds(cN + rq, RCH), :], abuf)

        @pl.loop(0, RCH)
        def _(i):
            si = sinvsmem[q * RCH + i]
            for m in range(HALF // LANES):
                sl = pl.ds(m * LANES, LANES)
                abuf[i, sl] = abuf[i, sl] * si

        pltpu.sync_copy(abuf, t01d_hbm.at[pl.ds(cN + rq, RCH), :])
        pltpu.sync_copy(abuf, acc.at[pl.ds(rq, RCH), :])

    plsc.subcore_barrier()

    NHBM = 3

    def gath(sidx, cix, bank, gsem):
        tab = zout_hbm if cix >= GRP - NHBM else ztab
        return pltpu.make_async_copy(tab.at[sidx.at[cix]], bank, gsem)

    def scat(didx, cix, bank, ssem):
        return pltpu.make_async_copy(bank, acc.at[didx.at[cix]], ssem)

    def substep():

        banks = (bk0, bk1, bk2, bk3)
        gsems = (gsemA, gsemB, gsemC, gsemD)
        ssems = (ssemA, ssemB, ssemC, ssemD)

        def supergroup(t, p_s, p_d, q_s, q_d, first, last_sg):
            def preload():
                pltpu.async_copy(srcb_hbm.at[g0 + t + 1], q_s, isem)
                pltpu.async_copy(dstb_hbm.at[g0 + t + 1], q_d, isem)

            def wait_preload():
                pltpu.make_async_copy(srcb_hbm.at[g0 + t + 1], q_s, isem).wait()
                pltpu.make_async_copy(dstb_hbm.at[g0 + t + 1], q_d, isem).wait()

            @pl.loop(GRP - NHBM, GRP)
            def _(j):
                for m in range(CHUNK // LANES):
                    sl = pl.ds(m * LANES, LANES)
                    p_s[j, sl] = p_s[j, sl] + cN

            for cix in range(GRP):
                b = cix % 4
                if cix < 4:
                    if first is None:
                        scat(p_d, cix, banks[b], ssems[b]).wait()
                    else:
                        @pl.when(jnp.logical_not(first))
                        def _():
                            scat(p_d, cix, banks[b], ssems[b]).wait()
                else:
                    scat(p_d, cix, banks[b], ssems[b]).wait()
                if cix >= GRP - NHBM:
                    pltpu.async_copy(zout_hbm.at[p_s.at[cix]], banks[b],
                                     gsems[b])
                else:
                    pltpu.async_copy(ztab.at[p_s.at[cix]], banks[b], gsems[b])
                if cix == 3:
                    if last_sg is None:
                        preload()
                    else:
                        @pl.when(jnp.logical_not(last_sg))
                        def _():
                            preload()
                if cix >= 2:
                    w = cix - 2
                    gath(p_s, w, banks[w % 4], gsems[w % 4]).wait()
                    pltpu.async_copy(banks[w % 4], acc.at[p_d.at[w]],
                                     ssems[w % 4], add=True)
            for w in (GRP - 2, GRP - 1):
                gath(p_s, w, banks[w % 4], gsems[w % 4]).wait()
                pltpu.async_copy(banks[w % 4], acc.at[p_d.at[w]],
                                 ssems[w % 4], add=True)
            if last_sg is None:
                wait_preload()
            else:
                @pl.when(jnp.logical_not(last_sg))
                def _():
                    wait_preload()

        pltpu.sync_copy(srcb_hbm.at[g0], sidx0)
        pltpu.sync_copy(dstb_hbm.at[g0], didx0)

        @pl.loop(0, NSG // 2)
        def _(u):
            t = 2 * u
            supergroup(t, sidx0, didx0, sidx1, didx1, u == 0, None)
            supergroup(t + 1, sidx1, didx1, sidx0, didx0, None,
                       u == NSG // 2 - 1)

        for b in range(4):
            scat(didx0, b, banks[b], ssems[b]).wait()
        plsc.subcore_barrier()

        for q in range(ROWS_PT // RCH):
            rq = r0 + q * RCH
            pltpu.sync_copy(acc.at[pl.ds(rq, RCH), :], abuf)
            pltpu.async_copy(t01d_hbm.at[pl.ds(cN + rq, RCH), :],
                             acc.at[pl.ds(rq, RCH), :], isem)

            @pl.loop(0, RCH)
            def _(i):
                rd = s9smem[q * RCH + i]
                for m in range(HALF // LANES):
                    sl = pl.ds(m * LANES, LANES)
                    abuf[i, sl] = abuf[i, sl] * rd

            pltpu.sync_copy(abuf, ztab.at[pl.ds(rq, RCH), :])
            pltpu.sync_copy(abuf, zout_hbm.at[pl.ds(cN + rq, RCH), :])
        for q in range(ROWS_PT // RCH):
            rq = r0 + q * RCH
            pltpu.make_async_copy(t01d_hbm.at[pl.ds(cN + rq, RCH), :],
                                  acc.at[pl.ds(rq, RCH), :], isem).wait()
        plsc.subcore_barrier()

    @pl.loop(0, K)
    def _(k):
        substep()


def kernel(X, edge_index, type_nodes, W_enc, b_enc, W_lin, b_lin):
    maskf = type_nodes.astype(jnp.float32)
    ts, t01, means, std = pl.pallas_call(
        _tc_pre,
        out_shape=[
            jax.ShapeDtypeStruct((2, NP, HALF), jnp.float32),
            jax.ShapeDtypeStruct((2, NP, HALF), jnp.float32),
            jax.ShapeDtypeStruct((T, D), jnp.float32),
            jax.ShapeDtypeStruct((T, D), jnp.float32),
        ],
        compiler_params=pltpu.CompilerParams(vmem_limit_bytes=100 * 2**20),
    )(X, W_enc, b_enc, maskf)

    ts2 = ts.reshape(2 * NP, HALF)
    t012 = t01.reshape(2 * NP, HALF)

    src = edge_index[0]
    dst = edge_index[1]
    srcb = jnp.pad(src, (0, E_PAD - E)).reshape(NS * NSG, GRP, CHUNK)
    dstb = jnp.pad(dst, (0, E_PAD - E), constant_values=N).reshape(NS * NSG, GRP, CHUNK)

    mesh = plsc.VectorSubcoreMesh(core_axis_name="c", subcore_axis_name="s",
                                  num_cores=NC, num_subcores=NS)
    zfin, _ = pl.kernel(
        _sc_diffuse,
        out_type=[jax.ShapeDtypeStruct((2 * NP, HALF), jnp.float32),
                  jax.ShapeDtypeStruct((2 * NP, HALF), jnp.float32)],
        mesh=mesh,
        compiler_params=pltpu.CompilerParams(use_tc_tiling_on_sc=False,
                                            needs_layout_passes=False),
        scratch_types=[
            pltpu.VMEM((GRP, CHUNK), jnp.int32),
            pltpu.VMEM((GRP, CHUNK), jnp.int32),
            pltpu.VMEM((GRP, CHUNK), jnp.int32),
            pltpu.VMEM((GRP, CHUNK), jnp.int32),
            pltpu.VMEM((CHUNK, HALF), jnp.float32),
            pltpu.VMEM((CHUNK, HALF), jnp.float32),
            pltpu.VMEM((CHUNK, HALF), jnp.float32),
            pltpu.VMEM((CHUNK, HALF), jnp.float32),
            pltpu.VMEM((RCH, HALF), jnp.float32),
            pltpu.VMEM_SHARED((NP, HALF), jnp.float32),
            pltpu.VMEM_SHARED((NP, HALF), jnp.float32),
            pltpu.SMEM((ROWS_PT,), jnp.float32),
            pltpu.SMEM((ROWS_PT,), jnp.float32),
        ] + [pltpu.SemaphoreType.DMA] * 9,
    )(ts2, t012, srcb, dstb)

    out = pl.pallas_call(
        _tc_post,
        out_shape=jax.ShapeDtypeStruct((N, D), jnp.float32),
        compiler_params=pltpu.CompilerParams(vmem_limit_bytes=100 * 2**20),
    )(zfin.reshape(2, NP, HALF), maskf, means, std, W_lin, b_lin)
    return out


# speedup vs baseline: 1.2663x; 1.2663x over previous
import jax
import jax.numpy as jnp
from jax import lax
from jax.experimental import pallas as pl
from jax.experimental.pallas import tpu as pltpu
from jax.experimental.pallas import tpu_sc as plsc

N = 10000
E = 320000
D = 128
HALF = 64
T = 4
K = 10
ALPHA = 0.1

NC = 2
NS = 16
LANES = 16
CHUNK = 128
NP = 10240
NCH = 160
GRP = 8
NSG = NCH // GRP
HG = 2
EPT = NCH * CHUNK
E_PAD = NS * EPT
ROWS_PT = NP // NS
RCH = 128
CCH = 64


def _tc_pre(x_ref, w_ref, b_ref, m_ref, ts_ref, t01_ref, mean_ref, std_ref):
    x = x_ref[...]
    w = w_ref[...]
    b = b_ref[...]
    mask = m_ref[...]
    h = jnp.dot(x, w, preferred_element_type=jnp.float32,
                precision=lax.Precision.HIGHEST) + b[None, :]
    nrm = jnp.sqrt(jnp.sum(h * h, axis=1, keepdims=True))
    h = h / jnp.maximum(nrm, 1e-12)
    counts = jnp.sum(mask, axis=1)
    inv_c = 1.0 / counts
    means = jnp.dot(mask, h, preferred_element_type=jnp.float32,
                    precision=lax.Precision.HIGHEST) * inv_c[:, None]
    m2 = jnp.dot(mask, h * h, preferred_element_type=jnp.float32,
                 precision=lax.Precision.HIGHEST) * inv_c[:, None]
    var = m2 - means * means
    std = jnp.sqrt(jnp.maximum(var, 0.0))
    std = std * jnp.sqrt(counts)[:, None] + 1e-9
    mu = jnp.zeros((N, D), jnp.float32)
    sg = jnp.zeros((N, D), jnp.float32)
    for t in range(T):
        mt = mask[t][:, None]
        mu = mu + mt * means[t][None, :]
        sg = sg + mt * std[t][None, :]
    tilde = (h - mu) / sg
    pad = jnp.zeros((NP - N, HALF), jnp.float32)
    for half in range(2):
        th = tilde[:, half * HALF:(half + 1) * HALF]
        ts_ref[half, pl.ds(0, N)] = th
        ts_ref[half, pl.ds(N, NP - N)] = pad
        t01_ref[half, pl.ds(0, N)] = ALPHA * th
        t01_ref[half, pl.ds(N, NP - N)] = pad
    mean_ref[...] = means
    std_ref[...] = std


def _tc_post(z_ref, m_ref, mean_ref, std_ref, w_ref, b_ref, o_ref):
    mask = m_ref[...]
    means = mean_ref[...]
    std = std_ref[...]
    mu = jnp.zeros((N, D), jnp.float32)
    sg = jnp.zeros((N, D), jnp.float32)
    for t in range(T):
        mt = mask[t][:, None]
        mu = mu + mt * means[t][None, :]
        sg = sg + mt * std[t][None, :]
    z = jnp.concatenate([z_ref[0, pl.ds(0, N)], z_ref[1, pl.ds(0, N)]], axis=1)
    z = z * sg + mu
    o = jnp.dot(z, w_ref[...], preferred_element_type=jnp.float32,
                precision=lax.Precision.HIGHEST) + b_ref[...][None, :]
    nrm = jnp.sqrt(jnp.sum(o * o, axis=1, keepdims=True))
    o_ref[...] = o / jnp.maximum(nrm, 1e-12)


def _fill_rows(ref, nrows, ncols16, value):
    v = jnp.full((LANES,), value, jnp.float32)

    @pl.loop(0, nrows)
    def _(i):
        for m in range(ncols16):
            ref[i, pl.ds(m * LANES, LANES)] = v


def _sc_diffuse(ts_hbm, t01_hbm, srcb_hbm, dstb_hbm,
                zout_hbm, t01d_hbm,
                sidx0, sidx1, didx0, didx1, bk0, bk1, bk2, bk3, abuf,
                ztab, acc, s9smem, sinvsmem,
                gsemA, gsemB, gsemC, gsemD, ssemA, ssemB, ssemC, ssemD, isem):
    c = lax.axis_index("c")
    s = lax.axis_index("s")
    cN = c * NP
    r0 = s * ROWS_PT
    g0 = s * NSG

    def zero_acc():
        _fill_rows(bk0, RCH, HALF // LANES, 0.0)
        for q in range(ROWS_PT // RCH):
            pltpu.sync_copy(bk0, acc.at[pl.ds(r0 + q * RCH, RCH), :])
        plsc.subcore_barrier()

    for q in range(ROWS_PT // RCH):
        pltpu.sync_copy(ts_hbm.at[pl.ds(cN + r0 + q * RCH, RCH), :],
                        ztab.at[pl.ds(r0 + q * RCH, RCH), :])

    zero_acc()
    _fill_rows(bk0, RCH, HALF // LANES, 1.0)

    @pl.loop(0, NSG)
    def _(t):
        pltpu.sync_copy(dstb_hbm.at[g0 + t], didx0)
        for b in range(GRP):
            pltpu.sync_copy(bk0, acc.at[didx0.at[b]], add=True)

    plsc.subcore_barrier()
    for q in range(ROWS_PT // RCH):
        pltpu.sync_copy(acc.at[pl.ds(r0 + q * RCH, RCH), :], abuf)

        @pl.loop(0, RCH)
        def _(i):
            v = jnp.maximum(abuf[i, pl.ds(0, LANES)], 1.0)
            s9smem[q * RCH + i] = jnp.max((1.0 - ALPHA) / v)
            sinvsmem[q * RCH + i] = jnp.max(v * (1.0 / (1.0 - ALPHA)))

    for q in range(ROWS_PT // RCH):
        rq = r0 + q * RCH
        pltpu.sync_copy(t01_hbm.at[pl.ds(cN + rq, RCH), :], abuf)

        @pl.loop(0, RCH)
        def _(i):
            si = sinvsmem[q * RCH + i]
            for m in range(HALF // LANES):
                sl = pl.ds(m * LANES, LANES)
                abuf[i, sl] = abuf[i, sl] * si

        pltpu.sync_copy(abuf, t01d_hbm.at[pl.ds(cN + rq, RCH), :])
        pltpu.sync_copy(abuf, acc.at[pl.ds(rq, RCH), :])

    plsc.subcore_barrier()

    def gath(sidx, cix, bank, gsem):
        return pltpu.make_async_copy(ztab.at[sidx.at[cix]], bank, gsem)

    def scat(didx, cix, bank, ssem):
        return pltpu.make_async_copy(bank, acc.at[didx.at[cix]], ssem)

    def substep():

        banks = (bk0, bk1, bk2, bk3)
        gsems = (gsemA, gsemB, gsemC, gsemD)
        ssems = (ssemA, ssemB, ssemC, ssemD)

        def supergroup(t, p_s, p_d, q_s, q_d, first, last_sg):
            def preload():
                pltpu.async_copy(srcb_hbm.at[g0 + t + 1], q_s, isem)
                pltpu.async_copy(dstb_hbm.at[g0 + t + 1], q_d, isem)

            def wait_preload():
                pltpu.make_async_copy(srcb_hbm.at[g0 + t + 1], q_s, isem).wait()
                pltpu.make_async_copy(dstb_hbm.at[g0 + t + 1], q_d, isem).wait()

            for cix in range(GRP):
                b = cix % 4
                if cix < 4:
                    if first is None:
                        scat(p_d, cix, banks[b], ssems[b]).wait()
                    else:
                        @pl.when(jnp.logical_not(first))
                        def _():
                            scat(p_d, cix, banks[b], ssems[b]).wait()
                else:
                    scat(p_d, cix, banks[b], ssems[b]).wait()
                pltpu.async_copy(ztab.at[p_s.at[cix]], banks[b], gsems[b])
                if cix == 3:
                    if last_sg is None:
                        preload()
                    else:
                        @pl.when(jnp.logical_not(last_sg))
                        def _():
                            preload()
                if cix >= 2:
                    w = cix - 2
                    gath(p_s, w, banks[w % 4], gsems[w % 4]).wait()
                    pltpu.async_copy(banks[w % 4], acc.at[p_d.at[w]],
                                     ssems[w % 4], add=True)
            for w in (GRP - 2, GRP - 1):
                gath(p_s, w, banks[w % 4], gsems[w % 4]).wait()
                pltpu.async_copy(banks[w % 4], acc.at[p_d.at[w]],
                                 ssems[w % 4], add=True)
            if last_sg is None:
                wait_preload()
            else:
                @pl.when(jnp.logical_not(last_sg))
                def _():
                    wait_preload()

        pltpu.sync_copy(srcb_hbm.at[g0], sidx0)
        pltpu.sync_copy(dstb_hbm.at[g0], didx0)

        @pl.loop(0, NSG // 2)
        def _(u):
            t = 2 * u
            supergroup(t, sidx0, didx0, sidx1, didx1, u == 0, None)
            supergroup(t + 1, sidx1, didx1, sidx0, didx0, None,
                       u == NSG // 2 - 1)

        for b in range(4):
            scat(didx0, b, banks[b], ssems[b]).wait()
        plsc.subcore_barrier()

        for q in range(ROWS_PT // RCH):
            rq = r0 + q * RCH
            pltpu.sync_copy(acc.at[pl.ds(rq, RCH), :], abuf)
            pltpu.async_copy(t01d_hbm.at[pl.ds(cN + rq, RCH), :],
                             acc.at[pl.ds(rq, RCH), :], isem)

            @pl.loop(0, RCH)
            def _(i):
                rd = s9smem[q * RCH + i]
                for m in range(HALF // LANES):
                    sl = pl.ds(m * LANES, LANES)
                    abuf[i, sl] = abuf[i, sl] * rd

            pltpu.sync_copy(abuf, ztab.at[pl.ds(rq, RCH), :])
            pltpu.sync_copy(abuf, zout_hbm.at[pl.ds(cN + rq, RCH), :])
        for q in range(ROWS_PT // RCH):
            rq = r0 + q * RCH
            pltpu.make_async_copy(t01d_hbm.at[pl.ds(cN + rq, RCH), :],
                                  acc.at[pl.ds(rq, RCH), :], isem).wait()
        plsc.subcore_barrier()

    @pl.loop(0, K)
    def _(k):
        substep()


def kernel(X, edge_index, type_nodes, W_enc, b_enc, W_lin, b_lin):
    maskf = type_nodes.astype(jnp.float32)
    ts, t01, means, std = pl.pallas_call(
        _tc_pre,
        out_shape=[
            jax.ShapeDtypeStruct((2, NP, HALF), jnp.float32),
            jax.ShapeDtypeStruct((2, NP, HALF), jnp.float32),
            jax.ShapeDtypeStruct((T, D), jnp.float32),
            jax.ShapeDtypeStruct((T, D), jnp.float32),
        ],
        compiler_params=pltpu.CompilerParams(vmem_limit_bytes=100 * 2**20),
    )(X, W_enc, b_enc, maskf)

    ts2 = ts.reshape(2 * NP, HALF)
    t012 = t01.reshape(2 * NP, HALF)

    src = edge_index[0]
    dst = edge_index[1]
    srcb = jnp.pad(src, (0, E_PAD - E)).reshape(NS * NSG, GRP, CHUNK)
    dstb = jnp.pad(dst, (0, E_PAD - E), constant_values=N).reshape(NS * NSG, GRP, CHUNK)

    mesh = plsc.VectorSubcoreMesh(core_axis_name="c", subcore_axis_name="s",
                                  num_cores=NC, num_subcores=NS)
    zfin, _ = pl.kernel(
        _sc_diffuse,
        out_type=[jax.ShapeDtypeStruct((2 * NP, HALF), jnp.float32),
                  jax.ShapeDtypeStruct((2 * NP, HALF), jnp.float32)],
        mesh=mesh,
        compiler_params=pltpu.CompilerParams(use_tc_tiling_on_sc=False,
                                            needs_layout_passes=False),
        scratch_types=[
            pltpu.VMEM((GRP, CHUNK), jnp.int32),
            pltpu.VMEM((GRP, CHUNK), jnp.int32),
            pltpu.VMEM((GRP, CHUNK), jnp.int32),
            pltpu.VMEM((GRP, CHUNK), jnp.int32),
            pltpu.VMEM((CHUNK, HALF), jnp.float32),
            pltpu.VMEM((CHUNK, HALF), jnp.float32),
            pltpu.VMEM((CHUNK, HALF), jnp.float32),
            pltpu.VMEM((CHUNK, HALF), jnp.float32),
            pltpu.VMEM((RCH, HALF), jnp.float32),
            pltpu.VMEM_SHARED((NP, HALF), jnp.float32),
            pltpu.VMEM_SHARED((NP, HALF), jnp.float32),
            pltpu.SMEM((ROWS_PT,), jnp.float32),
            pltpu.SMEM((ROWS_PT,), jnp.float32),
        ] + [pltpu.SemaphoreType.DMA] * 9,
    )(ts2, t012, srcb, dstb)

    out = pl.pallas_call(
        _tc_post,
        out_shape=jax.ShapeDtypeStruct((N, D), jnp.float32),
        compiler_params=pltpu.CompilerParams(vmem_limit_bytes=100 * 2**20),
    )(zfin.reshape(2, NP, HALF), maskf, means, std, W_lin, b_lin)
    return out
